# baseline (speedup 1.0000x reference)
import functools

import jax
import jax.numpy as jnp
from jax import lax
from jax.experimental import pallas as pl
from jax.experimental.pallas import tpu as pltpu
from jax.experimental.pallas import tpu_sc as plsc

B = 8
N_PER = 1250
E_PER = 40000
LM = 1024
HID = 128
NREL = 38
HOPS = 5
NAUG = N_PER + 1
NTOT = B * NAUG

NR = 1264
TPAD = 48
AEOFF = 128
AE_W = AEOFF + NR
EP = 47104
NTILE = 16
ET = EP // NTILE
NCH = ET // 128
GPC = 4
MW = NR * NR
SW = MW // NTILE
ZCS = 2944
ZNC = SW // ZCS
ZLAST = SW - ZNC * ZCS
SCS = 8192
SNC = SW // SCS
SLAST = SW - SNC * SCS
CW = NR * TPAD
CR = GPC * CW + 128
CSW = CR // NTILE
CP = CW // NTILE

@functools.lru_cache(maxsize=None)
def _mesh():
    return plsc.VectorSubcoreMesh(core_axis_name="c", subcore_axis_name="s")


def _build_edges(node_type, edge_index, edge_type):
    nt = node_type.reshape(B, N_PER).astype(jnp.int32)
    j = jnp.arange(1, NAUG, dtype=jnp.int32)
    isq = nt == 0
    isa = nt == 1
    jj = jnp.broadcast_to(j[None], (B, N_PER))
    zero = jnp.zeros_like(jj)
    dum = jnp.full_like(jj, NAUG)
    s1, d1, t1 = zero, jnp.where(isq, jj, dum), jnp.where(isq, 38, 63)
    s2, d2, t2 = zero, jnp.where(isa, jj, dum), jnp.where(isa, 39, 63)
    s3, d3, t3 = jnp.where(isq, jj, zero), jnp.where(isq, zero, dum), jnp.where(isq, 40, 63)
    s4, d4, t4 = jnp.where(isa, jj, zero), jnp.where(isa, zero, dum), jnp.where(isa, 41, 63)
    ei = edge_index.reshape(2, B, E_PER).astype(jnp.int32)
    boff = (jnp.arange(B, dtype=jnp.int32) * N_PER)[None, :, None]
    loc = ei - boff + 1
    rt = edge_type.reshape(B, E_PER).astype(jnp.int32)
    n = jnp.broadcast_to(jnp.arange(NAUG, dtype=jnp.int32)[None], (B, NAUG))
    src = jnp.concatenate([s1, s2, s3, s4, loc[0], n], axis=1)
    dst = jnp.concatenate([d1, d2, d3, d4, loc[1], n], axis=1)
    ae = jnp.concatenate([t1, t2, t3, t4, rt, n + AEOFF], axis=1)
    pad = EP - src.shape[1]
    src = jnp.pad(src, ((0, 0), (0, pad)))
    dst = jnp.pad(dst, ((0, 0), (0, pad)), constant_values=NAUG)
    ae = jnp.pad(ae, ((0, 0), (0, pad)), constant_values=63)
    return (src.reshape(B, NTILE, ET), dst.reshape(B, NTILE, ET),
            ae.reshape(B, NTILE, ET))


def _sc_pre_body(ed, ea, c_out, edd, eda, vals1, idxb, zb, csp):
    t = lax.axis_index("s")
    c = lax.axis_index("c")

    def zf(i, carry):
        zb[pl.ds(i * 16, 16)] = jnp.zeros((16,), jnp.float32)
        return carry
    lax.fori_loop(0, (CSW + 15) // 16, zf, 0)

    def of(i, carry):
        vals1[pl.ds(i * 16, 16)] = jnp.ones((16,), jnp.float32)
        return carry
    lax.fori_loop(0, ET // 16, of, 0)

    pltpu.sync_copy(zb.at[pl.ds(0, CSW)], csp.at[pl.ds(t * CSW, CSW)])
    plsc.subcore_barrier()

    for r in range(GPC):
        g = c * GPC + r
        eoff = (g * NTILE + t) * ET
        pltpu.sync_copy(ed.at[pl.ds(eoff, ET)], edd)
        pltpu.sync_copy(ea.at[pl.ds(eoff, ET)], eda)

        def grp(gi, carry):
            i = gi * 16
            d16 = edd[pl.ds(i, 16)]
            a16 = eda[pl.ds(i, 16)]
            real = a16 < 42
            cidx = jnp.where(real, (d16 + r * NR) * TPAD + a16,
                             jnp.int32(GPC * CW))
            idxb[pl.ds(i, 16)] = cidx
            return carry
        lax.fori_loop(0, ET // 16, grp, 0)
        pltpu.sync_copy(vals1, csp.at[idxb], add=True)

    plsc.subcore_barrier()
    for r in range(GPC):
        g = c * GPC + r
        pltpu.sync_copy(csp.at[pl.ds(r * CW + t * CP, CP)],
                        zb.at[pl.ds(0, CP)])
        pltpu.sync_copy(zb.at[pl.ds(0, CP)],
                        c_out.at[pl.ds(g * CW + t * CP, CP)])


@functools.lru_cache(maxsize=None)
def _sc_pre():
    return pl.kernel(
        _sc_pre_body,
        out_type=jax.ShapeDtypeStruct((B * CW,), jnp.float32),
        mesh=_mesh(),
        compiler_params=pltpu.CompilerParams(needs_layout_passes=False),
        scratch_types=[
            pltpu.VMEM((ET,), jnp.int32),
            pltpu.VMEM((ET,), jnp.int32),
            pltpu.VMEM((ET,), jnp.float32),
            pltpu.VMEM((ET,), jnp.int32),
            pltpu.VMEM((((CSW + 15) // 16) * 16,), jnp.float32),
            pltpu.VMEM_SHARED((CR,), jnp.float32),
        ],
    )


def _sc_hop_body(asrc, adst, aetab, es, ed, ea, m_out,
                 eds, edd, eda, asv, adv, aev, vals, idxb, z2, stg, msp):
    t = lax.axis_index("s")
    c = lax.axis_index("c")

    def zf(i, carry):
        z2[pl.ds(i * 16, 16)] = jnp.zeros((16,), jnp.float32)
        return carry
    lax.fori_loop(0, (ZCS + 15) // 16, zf, 0)

    def zero_msp():
        for k in range(ZNC):
            pltpu.sync_copy(z2.at[pl.ds(0, ZCS)],
                            msp.at[pl.ds(t * SW + k * ZCS, ZCS)])
        pltpu.sync_copy(z2.at[pl.ds(0, ZLAST)],
                        msp.at[pl.ds(t * SW + ZNC * ZCS, ZLAST)])

    zero_msp()
    plsc.subcore_barrier()

    for r in range(GPC):
        g = c * GPC + r
        pltpu.sync_copy(asrc.at[pl.ds(g * NR, NR)], asv)
        pltpu.sync_copy(adst.at[pl.ds(g * NR, NR)], adv)
        pltpu.sync_copy(aetab.at[pl.ds(g * AE_W, AE_W)], aev)
        eoff = (g * NTILE + t) * ET
        pltpu.sync_copy(es.at[pl.ds(eoff, ET)], eds)
        pltpu.sync_copy(ed.at[pl.ds(eoff, ET)], edd)
        pltpu.sync_copy(ea.at[pl.ds(eoff, ET)], eda)

        def grp(gi, carry):
            i = gi * 16
            s16 = eds[pl.ds(i, 16)]
            d16 = edd[pl.ds(i, 16)]
            a16 = eda[pl.ds(i, 16)]
            av = plsc.load_gather(asv, [s16])
            bv = plsc.load_gather(adv, [d16])
            ev = plsc.load_gather(aev, [a16])
            al = av + bv + ev
            al = jnp.where(al > 0, al, al * jnp.float32(0.2))
            ex = jnp.exp(al)
            vals[pl.ds(i, 16)] = ex
            idxb[pl.ds(i, 16)] = d16 * NR + s16
            return carry
        lax.fori_loop(0, ET // 16, grp, 0)
        pltpu.sync_copy(vals, msp.at[idxb], add=True)

        plsc.subcore_barrier()
        for k in range(SNC + 1):
            csz = SCS if k < SNC else SLAST
            pltpu.sync_copy(msp.at[pl.ds(t * SW + k * SCS, csz)],
                            stg.at[pl.ds(0, csz)])
            pltpu.sync_copy(stg.at[pl.ds(0, csz)],
                            m_out.at[pl.ds(g * MW + t * SW + k * SCS, csz)])
        plsc.subcore_barrier()
        if r < GPC - 1:
            pltpu.sync_copy(z2.at[pl.ds(0, ET)], msp.at[idxb])
        plsc.subcore_barrier()


@functools.lru_cache(maxsize=None)
def _sc_hop():
    return pl.kernel(
        _sc_hop_body,
        out_type=jax.ShapeDtypeStruct((B * MW,), jnp.float32),
        mesh=_mesh(),
        compiler_params=pltpu.CompilerParams(needs_layout_passes=False),
        scratch_types=[
            pltpu.VMEM((ET,), jnp.int32),
            pltpu.VMEM((ET,), jnp.int32),
            pltpu.VMEM((ET,), jnp.int32),
            pltpu.VMEM((NR,), jnp.float32),
            pltpu.VMEM((NR,), jnp.float32),
            pltpu.VMEM((AE_W,), jnp.float32),
            pltpu.VMEM((ET,), jnp.float32),
            pltpu.VMEM((ET,), jnp.int32),
            pltpu.VMEM((((ZCS + 15) // 16) * 16,), jnp.float32),
            pltpu.VMEM((SCS,), jnp.float32),
            pltpu.VMEM_SHARED((MW,), jnp.float32),
        ],
    )


def _tmpb_body(lm_ref, wb_ref, out_ref):
    lm = lm_ref[...]
    for i in range(8):
        out_ref[:, i, :] = jnp.dot(lm, wb_ref[i],
                                   preferred_element_type=jnp.float32)


def _tc_tmpb(lm, w_bil):
    return pl.pallas_call(
        _tmpb_body,
        grid=(HID // 8,),
        in_specs=[
            pl.BlockSpec((B, LM), lambda k: (0, 0)),
            pl.BlockSpec((8, LM, HID), lambda k: (k, 0, 0)),
        ],
        out_specs=pl.BlockSpec((B, 8, HID), lambda k: (0, k, 0)),
        out_shape=jax.ShapeDtypeStruct((B, HID, HID), jnp.float32),
    )(lm, w_bil)


def _init_body(lm_ref, wlm_ref, blm_ref, ne_ref, tb_ref, bbil_ref, out_ref):
    ctx = jnp.dot(lm_ref[0], wlm_ref[...],
                  preferred_element_type=jnp.float32) + blm_ref[...]
    emb = jnp.concatenate(
        [ctx, ne_ref[0], jnp.zeros((NR - NAUG, HID), jnp.float32)], axis=0)
    nd = lax.dot_general(emb, tb_ref[0], (((1,), (1,)), ((), ())),
                         preferred_element_type=jnp.float32) + bbil_ref[...]
    row = lax.broadcasted_iota(jnp.int32, (NR, 1), 0)
    out_ref[0] = jnp.where(row < NAUG, nd, 0.0)


def _tc_init(lm, w_lm, b_lm, node_emb3, tmpb, b_bil):
    return pl.pallas_call(
        _init_body,
        grid=(B,),
        in_specs=[
            pl.BlockSpec((1, 1, LM), lambda b: (b, 0, 0)),
            pl.BlockSpec((LM, HID), lambda b: (0, 0)),
            pl.BlockSpec((1, HID), lambda b: (0, 0)),
            pl.BlockSpec((1, N_PER, HID), lambda b: (b, 0, 0)),
            pl.BlockSpec((1, HID, HID), lambda b: (b, 0, 0)),
            pl.BlockSpec((1, HID), lambda b: (0, 0)),
        ],
        out_specs=pl.BlockSpec((1, NR, HID), lambda b: (b, 0, 0)),
        out_shape=jax.ShapeDtypeStruct((B, NR, HID), jnp.float32),
    )(lm.reshape(B, 1, LM), w_lm, b_lm.reshape(1, HID), node_emb3, tmpb,
      b_bil.reshape(1, HID))


def _selfattr_body(c_ref, tab_ref, out_ref):
    cm = c_ref[0]
    cnt = jnp.sum(cm, axis=1, keepdims=True)
    s = jnp.dot(cm, tab_ref[...], preferred_element_type=jnp.float32)
    out_ref[0] = s / jnp.maximum(cnt, 1.0)


def _tc_selfattr(cmat, tab):
    return pl.pallas_call(
        _selfattr_body,
        grid=(B,),
        in_specs=[
            pl.BlockSpec((1, NR, TPAD), lambda b: (b, 0, 0)),
            pl.BlockSpec((TPAD, HID), lambda b: (0, 0)),
        ],
        out_specs=pl.BlockSpec((1, NR, HID), lambda b: (b, 0, 0)),
        out_shape=jax.ShapeDtypeStruct((B, NR, HID), jnp.float32),
    )(cmat, tab)


def _head_body(nd_ref, w_ref, asr_ref, adr_ref, le_ref, aer_ref, tab_ref,
               sa_ref, x_ref, as_ref, ad_ref, ae_ref):
    x = jnp.dot(nd_ref[0], w_ref[...], preferred_element_type=jnp.float32)
    x_ref[0] = x
    as_ref[0, 0, :] = jnp.sum(x * asr_ref[...], axis=1)
    ad_ref[0, 0, :] = jnp.sum(x * adr_ref[...], axis=1)
    we = jnp.sum(le_ref[...] * aer_ref[...], axis=1, keepdims=True)
    avec = jnp.dot(tab_ref[...], we, preferred_element_type=jnp.float32)
    s_self = jnp.dot(sa_ref[0], we, preferred_element_type=jnp.float32)
    row = jnp.concatenate(
        [avec.T, jnp.zeros((1, AEOFF - TPAD), jnp.float32), s_self.T], axis=1)
    ae_ref[0] = row


def _tc_head(nodes, w, att_s, att_d, lin_e, att_e, tab, self_attr):
    return pl.pallas_call(
        _head_body,
        grid=(B,),
        in_specs=[
            pl.BlockSpec((1, NR, HID), lambda b: (b, 0, 0)),
            pl.BlockSpec((HID, HID), lambda b: (0, 0)),
            pl.BlockSpec((1, HID), lambda b: (0, 0)),
            pl.BlockSpec((1, HID), lambda b: (0, 0)),
            pl.BlockSpec((HID, HID), lambda b: (0, 0)),
            pl.BlockSpec((1, HID), lambda b: (0, 0)),
            pl.BlockSpec((TPAD, HID), lambda b: (0, 0)),
            pl.BlockSpec((1, NR, HID), lambda b: (b, 0, 0)),
        ],
        out_specs=[
            pl.BlockSpec((1, NR, HID), lambda b: (b, 0, 0)),
            pl.BlockSpec((1, 1, NR), lambda b: (b, 0, 0)),
            pl.BlockSpec((1, 1, NR), lambda b: (b, 0, 0)),
            pl.BlockSpec((1, 1, AE_W), lambda b: (b, 0, 0)),
        ],
        out_shape=[
            jax.ShapeDtypeStruct((B, NR, HID), jnp.float32),
            jax.ShapeDtypeStruct((B, 1, NR), jnp.float32),
            jax.ShapeDtypeStruct((B, 1, NR), jnp.float32),
            jax.ShapeDtypeStruct((B, 1, AE_W), jnp.float32),
        ],
    )(nodes, w, att_s.reshape(1, HID), att_d.reshape(1, HID), lin_e,
      att_e.reshape(1, HID), tab, self_attr)


def _comb_body(m_ref, x_ref, b_ref, out_ref):
    m = m_ref[0]
    agg = jnp.dot(m, x_ref[0], preferred_element_type=jnp.float32)
    den = jnp.sum(m, axis=1, keepdims=True)
    nd = jax.nn.gelu(agg / (den + 1e-16) + b_ref[...])
    row = lax.broadcasted_iota(jnp.int32, (NR, 1), 0)
    out_ref[0] = jnp.where(row < NAUG, nd, 0.0)


def _tc_comb(m, x, bias):
    return pl.pallas_call(
        _comb_body,
        grid=(B,),
        in_specs=[
            pl.BlockSpec((1, NR, NR), lambda b: (b, 0, 0)),
            pl.BlockSpec((1, NR, HID), lambda b: (b, 0, 0)),
            pl.BlockSpec((1, HID), lambda b: (0, 0)),
        ],
        out_specs=pl.BlockSpec((1, NR, HID), lambda b: (b, 0, 0)),
        out_shape=jax.ShapeDtypeStruct((B, NR, HID), jnp.float32),
    )(m, x, bias.reshape(1, HID))


def kernel(node_emb, node_type, edge_index, edge_type, lm_context, W_lm,
           b_lm, W_bil, b_bil, edge_table, gat_lin, gat_att_src, gat_att_dst,
           gat_lin_edge, gat_att_edge, gat_bias):
    es, ed, ea = _build_edges(node_type, edge_index, edge_type)
    tab = jnp.pad(edge_table, ((0, TPAD - NREL - 4), (0, 0)))

    cmat = _sc_pre()(ed.reshape(-1), ea.reshape(-1)).reshape(B, NR, TPAD)
    self_attr = _tc_selfattr(cmat, tab)

    tmpb = _tc_tmpb(lm_context, W_bil)
    nodes = _tc_init(lm_context, W_lm, b_lm, node_emb.reshape(B, N_PER, HID),
                     tmpb, b_bil)

    for l in range(HOPS):
        x, a_s, a_d, ae_tab = _tc_head(nodes, gat_lin[l], gat_att_src[l],
                                       gat_att_dst[l], gat_lin_edge[l],
                                       gat_att_edge[l], tab, self_attr)
        m = _sc_hop()(a_s.reshape(-1), a_d.reshape(-1), ae_tab.reshape(-1),
                      es.reshape(-1), ed.reshape(-1),
                      ea.reshape(-1)).reshape(B, NR, NR)
        nodes = _tc_comb(m, x, gat_bias[l])
    return nodes[:, 0, :]


# speedup vs baseline: 1.0367x; 1.0367x over previous
import functools

import jax
import jax.numpy as jnp
from jax import lax
from jax.experimental import pallas as pl
from jax.experimental.pallas import tpu as pltpu
from jax.experimental.pallas import tpu_sc as plsc

B = 8
N_PER = 1250
E_PER = 40000
LM = 1024
HID = 128
NREL = 38
HOPS = 5
NAUG = N_PER + 1
NTOT = B * NAUG

NR = 1264
TPAD = 48
AEOFF = 128
AE_W = AEOFF + NR
EP = 47104
NTILE = 16
ET = EP // NTILE
NCH = ET // 128
GPC = 4
MW = NR * NR
SW = MW // NTILE
ZCS = 2048
ZNC = SW // ZCS
ZLAST = SW - ZNC * ZCS
SCS = 9848
SNC = SW // SCS
SLAST = SW - SNC * SCS
CW = NR * TPAD
CR = GPC * CW + 128
CSW = CR // NTILE
CP = CW // NTILE

@functools.lru_cache(maxsize=None)
def _mesh():
    return plsc.VectorSubcoreMesh(core_axis_name="c", subcore_axis_name="s")


def _build_edges(node_type, edge_index, edge_type):
    nt = node_type.reshape(B, N_PER).astype(jnp.int32)
    j = jnp.arange(1, NAUG, dtype=jnp.int32)
    isq = nt == 0
    isa = nt == 1
    jj = jnp.broadcast_to(j[None], (B, N_PER))
    zero = jnp.zeros_like(jj)
    dum = jnp.full_like(jj, NAUG)
    s1, d1, t1 = zero, jnp.where(isq, jj, dum), jnp.where(isq, 38, 63)
    s2, d2, t2 = zero, jnp.where(isa, jj, dum), jnp.where(isa, 39, 63)
    s3, d3, t3 = jnp.where(isq, jj, zero), jnp.where(isq, zero, dum), jnp.where(isq, 40, 63)
    s4, d4, t4 = jnp.where(isa, jj, zero), jnp.where(isa, zero, dum), jnp.where(isa, 41, 63)
    ei = edge_index.reshape(2, B, E_PER).astype(jnp.int32)
    boff = (jnp.arange(B, dtype=jnp.int32) * N_PER)[None, :, None]
    loc = ei - boff + 1
    rt = edge_type.reshape(B, E_PER).astype(jnp.int32)
    n = jnp.broadcast_to(jnp.arange(NAUG, dtype=jnp.int32)[None], (B, NAUG))
    src = jnp.concatenate([s1, s2, s3, s4, loc[0], n], axis=1)
    dst = jnp.concatenate([d1, d2, d3, d4, loc[1], n], axis=1)
    ae = jnp.concatenate([t1, t2, t3, t4, rt, n + AEOFF], axis=1)
    pad = EP - src.shape[1]
    src = jnp.pad(src, ((0, 0), (0, pad)))
    dst = jnp.pad(dst, ((0, 0), (0, pad)), constant_values=NAUG)
    ae = jnp.pad(ae, ((0, 0), (0, pad)), constant_values=63)
    return (src.reshape(B, NTILE, ET), dst.reshape(B, NTILE, ET),
            ae.reshape(B, NTILE, ET))


def _sc_pre_body(ed, ea, c_out, edd, eda, vals1, idxb, zb, csp):
    t = lax.axis_index("s")
    c = lax.axis_index("c")

    def zf(i, carry):
        zb[pl.ds(i * 16, 16)] = jnp.zeros((16,), jnp.float32)
        return carry
    lax.fori_loop(0, (CSW + 15) // 16, zf, 0)

    def of(i, carry):
        vals1[pl.ds(i * 16, 16)] = jnp.ones((16,), jnp.float32)
        return carry
    lax.fori_loop(0, ET // 16, of, 0)

    pltpu.sync_copy(zb.at[pl.ds(0, CSW)], csp.at[pl.ds(t * CSW, CSW)])
    plsc.subcore_barrier()

    for r in range(GPC):
        g = c * GPC + r
        eoff = (g * NTILE + t) * ET
        pltpu.sync_copy(ed.at[pl.ds(eoff, ET)], edd)
        pltpu.sync_copy(ea.at[pl.ds(eoff, ET)], eda)

        def chunk(cc, carry):
            for v in range(8):
                i = cc * 128 + v * 16
                d16 = edd[pl.ds(i, 16)]
                a16 = eda[pl.ds(i, 16)]
                real = a16 < 42
                cidx = jnp.where(real, (d16 + r * NR) * TPAD + a16,
                                 jnp.int32(GPC * CW))
                idxb[cc, pl.ds(v * 16, 16)] = cidx
            pltpu.sync_copy(vals1.at[pl.ds(cc * 128, 128)],
                            csp.at[idxb.at[cc]], add=True)
            return carry
        lax.fori_loop(0, NCH, chunk, 0)

    plsc.subcore_barrier()
    for r in range(GPC):
        g = c * GPC + r
        pltpu.sync_copy(csp.at[pl.ds(r * CW + t * CP, CP)],
                        zb.at[pl.ds(0, CP)])
        pltpu.sync_copy(zb.at[pl.ds(0, CP)],
                        c_out.at[pl.ds(g * CW + t * CP, CP)])


@functools.lru_cache(maxsize=None)
def _sc_pre():
    return pl.kernel(
        _sc_pre_body,
        out_type=jax.ShapeDtypeStruct((B * CW,), jnp.float32),
        mesh=_mesh(),
        compiler_params=pltpu.CompilerParams(needs_layout_passes=False),
        scratch_types=[
            pltpu.VMEM((ET,), jnp.int32),
            pltpu.VMEM((ET,), jnp.int32),
            pltpu.VMEM((ET,), jnp.float32),
            pltpu.VMEM((NCH, 128), jnp.int32),
            pltpu.VMEM((((CSW + 15) // 16) * 16,), jnp.float32),
            pltpu.VMEM_SHARED((CR,), jnp.float32),
        ],
    )


def _sc_hop_body(asrc, adst, aetab, es, ed, ea, m_out,
                 eds, edd, eda, asv, adv, aev, vals, idxb, z2, stg, msp):
    t = lax.axis_index("s")
    c = lax.axis_index("c")

    def zf(i, carry):
        z2[pl.ds(i * 16, 16)] = jnp.zeros((16,), jnp.float32)
        return carry
    lax.fori_loop(0, (ZCS + 15) // 16, zf, 0)

    def zero_msp():
        for k in range(ZNC):
            pltpu.sync_copy(z2.at[pl.ds(0, ZCS)],
                            msp.at[pl.ds(t * SW + k * ZCS, ZCS)])
        pltpu.sync_copy(z2.at[pl.ds(0, ZLAST)],
                        msp.at[pl.ds(t * SW + ZNC * ZCS, ZLAST)])

    zero_msp()
    plsc.subcore_barrier()

    for r in range(GPC):
        g = c * GPC + r
        pltpu.sync_copy(asrc.at[pl.ds(g * NR, NR)], asv)
        pltpu.sync_copy(adst.at[pl.ds(g * NR, NR)], adv)
        pltpu.sync_copy(aetab.at[pl.ds(g * AE_W, AE_W)], aev)
        eoff = (g * NTILE + t) * ET
        pltpu.sync_copy(es.at[pl.ds(eoff, ET)], eds)
        pltpu.sync_copy(ed.at[pl.ds(eoff, ET)], edd)
        pltpu.sync_copy(ea.at[pl.ds(eoff, ET)], eda)

        def chunk(cc, carry):
            for v in range(8):
                i = cc * 128 + v * 16
                s16 = eds[pl.ds(i, 16)]
                d16 = edd[pl.ds(i, 16)]
                a16 = eda[pl.ds(i, 16)]
                av = plsc.load_gather(asv, [s16])
                bv = plsc.load_gather(adv, [d16])
                ev = plsc.load_gather(aev, [a16])
                al = av + bv + ev
                al = jnp.where(al > 0, al, al * jnp.float32(0.2))
                ex = jnp.exp(al)
                vals[pl.ds(i, 16)] = ex
                idxb[cc, pl.ds(v * 16, 16)] = d16 * NR + s16
            pltpu.sync_copy(vals.at[pl.ds(cc * 128, 128)],
                            msp.at[idxb.at[cc]], add=True)
            return carry
        lax.fori_loop(0, NCH, chunk, 0)

        plsc.subcore_barrier()
        for k in range(SNC + 1):
            csz = SCS if k < SNC else SLAST
            pltpu.sync_copy(msp.at[pl.ds(t * SW + k * SCS, csz)],
                            stg.at[pl.ds(0, csz)])
            pltpu.sync_copy(stg.at[pl.ds(0, csz)],
                            m_out.at[pl.ds(g * MW + t * SW + k * SCS, csz)])
        plsc.subcore_barrier()
        if r < GPC - 1:
            def zchunk(cc, carry):
                pltpu.sync_copy(z2.at[pl.ds(0, 128)], msp.at[idxb.at[cc]])
                return carry
            lax.fori_loop(0, NCH, zchunk, 0)
        plsc.subcore_barrier()


@functools.lru_cache(maxsize=None)
def _sc_hop():
    return pl.kernel(
        _sc_hop_body,
        out_type=jax.ShapeDtypeStruct((B * MW,), jnp.float32),
        mesh=_mesh(),
        compiler_params=pltpu.CompilerParams(needs_layout_passes=False),
        scratch_types=[
            pltpu.VMEM((ET,), jnp.int32),
            pltpu.VMEM((ET,), jnp.int32),
            pltpu.VMEM((ET,), jnp.int32),
            pltpu.VMEM((NR,), jnp.float32),
            pltpu.VMEM((NR,), jnp.float32),
            pltpu.VMEM((AE_W,), jnp.float32),
            pltpu.VMEM((ET,), jnp.float32),
            pltpu.VMEM((NCH, 128), jnp.int32),
            pltpu.VMEM((((ZCS + 15) // 16) * 16,), jnp.float32),
            pltpu.VMEM((SCS,), jnp.float32),
            pltpu.VMEM_SHARED((MW,), jnp.float32),
        ],
    )


def _tmpb_body(lm_ref, wb_ref, out_ref):
    lm = lm_ref[...]
    for i in range(8):
        out_ref[:, i, :] = jnp.dot(lm, wb_ref[i],
                                   preferred_element_type=jnp.float32)


def _tc_tmpb(lm, w_bil):
    return pl.pallas_call(
        _tmpb_body,
        grid=(HID // 8,),
        in_specs=[
            pl.BlockSpec((B, LM), lambda k: (0, 0)),
            pl.BlockSpec((8, LM, HID), lambda k: (k, 0, 0)),
        ],
        out_specs=pl.BlockSpec((B, 8, HID), lambda k: (0, k, 0)),
        out_shape=jax.ShapeDtypeStruct((B, HID, HID), jnp.float32),
    )(lm, w_bil)


def _init_body(lm_ref, wlm_ref, blm_ref, ne_ref, tb_ref, bbil_ref, out_ref):
    ctx = jnp.dot(lm_ref[0], wlm_ref[...],
                  preferred_element_type=jnp.float32) + blm_ref[...]
    emb = jnp.concatenate(
        [ctx, ne_ref[0], jnp.zeros((NR - NAUG, HID), jnp.float32)], axis=0)
    nd = lax.dot_general(emb, tb_ref[0], (((1,), (1,)), ((), ())),
                         preferred_element_type=jnp.float32) + bbil_ref[...]
    row = lax.broadcasted_iota(jnp.int32, (NR, 1), 0)
    out_ref[0] = jnp.where(row < NAUG, nd, 0.0)


def _tc_init(lm, w_lm, b_lm, node_emb3, tmpb, b_bil):
    return pl.pallas_call(
        _init_body,
        grid=(B,),
        in_specs=[
            pl.BlockSpec((1, 1, LM), lambda b: (b, 0, 0)),
            pl.BlockSpec((LM, HID), lambda b: (0, 0)),
            pl.BlockSpec((1, HID), lambda b: (0, 0)),
            pl.BlockSpec((1, N_PER, HID), lambda b: (b, 0, 0)),
            pl.BlockSpec((1, HID, HID), lambda b: (b, 0, 0)),
            pl.BlockSpec((1, HID), lambda b: (0, 0)),
        ],
        out_specs=pl.BlockSpec((1, NR, HID), lambda b: (b, 0, 0)),
        out_shape=jax.ShapeDtypeStruct((B, NR, HID), jnp.float32),
    )(lm.reshape(B, 1, LM), w_lm, b_lm.reshape(1, HID), node_emb3, tmpb,
      b_bil.reshape(1, HID))


def _selfattr_body(c_ref, tab_ref, out_ref):
    cm = c_ref[0]
    cnt = jnp.sum(cm, axis=1, keepdims=True)
    s = jnp.dot(cm, tab_ref[...], preferred_element_type=jnp.float32)
    out_ref[0] = s / jnp.maximum(cnt, 1.0)


def _tc_selfattr(cmat, tab):
    return pl.pallas_call(
        _selfattr_body,
        grid=(B,),
        in_specs=[
            pl.BlockSpec((1, NR, TPAD), lambda b: (b, 0, 0)),
            pl.BlockSpec((TPAD, HID), lambda b: (0, 0)),
        ],
        out_specs=pl.BlockSpec((1, NR, HID), lambda b: (b, 0, 0)),
        out_shape=jax.ShapeDtypeStruct((B, NR, HID), jnp.float32),
    )(cmat, tab)


def _head_body(nd_ref, w_ref, asr_ref, adr_ref, le_ref, aer_ref, tab_ref,
               sa_ref, x_ref, as_ref, ad_ref, ae_ref):
    x = jnp.dot(nd_ref[0], w_ref[...], preferred_element_type=jnp.float32)
    x_ref[0] = x
    as_ref[0, 0, :] = jnp.sum(x * asr_ref[...], axis=1)
    ad_ref[0, 0, :] = jnp.sum(x * adr_ref[...], axis=1)
    we = jnp.sum(le_ref[...] * aer_ref[...], axis=1, keepdims=True)
    avec = jnp.dot(tab_ref[...], we, preferred_element_type=jnp.float32)
    s_self = jnp.dot(sa_ref[0], we, preferred_element_type=jnp.float32)
    row = jnp.concatenate(
        [avec.T, jnp.zeros((1, AEOFF - TPAD), jnp.float32), s_self.T], axis=1)
    ae_ref[0] = row


def _tc_head(nodes, w, att_s, att_d, lin_e, att_e, tab, self_attr):
    return pl.pallas_call(
        _head_body,
        grid=(B,),
        in_specs=[
            pl.BlockSpec((1, NR, HID), lambda b: (b, 0, 0)),
            pl.BlockSpec((HID, HID), lambda b: (0, 0)),
            pl.BlockSpec((1, HID), lambda b: (0, 0)),
            pl.BlockSpec((1, HID), lambda b: (0, 0)),
            pl.BlockSpec((HID, HID), lambda b: (0, 0)),
            pl.BlockSpec((1, HID), lambda b: (0, 0)),
            pl.BlockSpec((TPAD, HID), lambda b: (0, 0)),
            pl.BlockSpec((1, NR, HID), lambda b: (b, 0, 0)),
        ],
        out_specs=[
            pl.BlockSpec((1, NR, HID), lambda b: (b, 0, 0)),
            pl.BlockSpec((1, 1, NR), lambda b: (b, 0, 0)),
            pl.BlockSpec((1, 1, NR), lambda b: (b, 0, 0)),
            pl.BlockSpec((1, 1, AE_W), lambda b: (b, 0, 0)),
        ],
        out_shape=[
            jax.ShapeDtypeStruct((B, NR, HID), jnp.float32),
            jax.ShapeDtypeStruct((B, 1, NR), jnp.float32),
            jax.ShapeDtypeStruct((B, 1, NR), jnp.float32),
            jax.ShapeDtypeStruct((B, 1, AE_W), jnp.float32),
        ],
    )(nodes, w, att_s.reshape(1, HID), att_d.reshape(1, HID), lin_e,
      att_e.reshape(1, HID), tab, self_attr)


def _comb_body(m_ref, x_ref, b_ref, out_ref):
    m = m_ref[0]
    agg = jnp.dot(m, x_ref[0], preferred_element_type=jnp.float32)
    den = jnp.sum(m, axis=1, keepdims=True)
    nd = jax.nn.gelu(agg / (den + 1e-16) + b_ref[...])
    row = lax.broadcasted_iota(jnp.int32, (NR, 1), 0)
    out_ref[0] = jnp.where(row < NAUG, nd, 0.0)


def _tc_comb(m, x, bias):
    return pl.pallas_call(
        _comb_body,
        grid=(B,),
        in_specs=[
            pl.BlockSpec((1, NR, NR), lambda b: (b, 0, 0)),
            pl.BlockSpec((1, NR, HID), lambda b: (b, 0, 0)),
            pl.BlockSpec((1, HID), lambda b: (0, 0)),
        ],
        out_specs=pl.BlockSpec((1, NR, HID), lambda b: (b, 0, 0)),
        out_shape=jax.ShapeDtypeStruct((B, NR, HID), jnp.float32),
    )(m, x, bias.reshape(1, HID))


def kernel(node_emb, node_type, edge_index, edge_type, lm_context, W_lm,
           b_lm, W_bil, b_bil, edge_table, gat_lin, gat_att_src, gat_att_dst,
           gat_lin_edge, gat_att_edge, gat_bias):
    es, ed, ea = _build_edges(node_type, edge_index, edge_type)
    tab = jnp.pad(edge_table, ((0, TPAD - NREL - 4), (0, 0)))

    cmat = _sc_pre()(ed.reshape(-1), ea.reshape(-1)).reshape(B, NR, TPAD)
    self_attr = _tc_selfattr(cmat, tab)

    tmpb = _tc_tmpb(lm_context, W_bil)
    nodes = _tc_init(lm_context, W_lm, b_lm, node_emb.reshape(B, N_PER, HID),
                     tmpb, b_bil)

    for l in range(HOPS):
        x, a_s, a_d, ae_tab = _tc_head(nodes, gat_lin[l], gat_att_src[l],
                                       gat_att_dst[l], gat_lin_edge[l],
                                       gat_att_edge[l], tab, self_attr)
        m = _sc_hop()(a_s.reshape(-1), a_d.reshape(-1), ae_tab.reshape(-1),
                      es.reshape(-1), ed.reshape(-1),
                      ea.reshape(-1)).reshape(B, NR, NR)
        nodes = _tc_comb(m, x, gat_bias[l])
    return nodes[:, 0, :]
